# baseline (speedup 1.0000x reference)
import functools

import jax
import jax.numpy as jnp
from jax import lax
from jax.experimental import pallas as pl
from jax.experimental.pallas import tpu as pltpu
from jax.experimental.pallas import tpu_sc as plsc

D = 64
WINDOW = 8
LANES = 16
BS = 128
GBT = 8
CHUNK = 256
CPG = GBT * BS // CHUNK
NBUF = 2


def kernel(input, table, b_embed):
    B, N = input.shape
    BN = B * N
    idx_t = input.T.reshape(BN).astype(jnp.int32)

    info = plsc.get_sparse_core_info()
    num_workers = info.num_cores * info.num_subcores
    n_bt = B // BS
    n_grp = n_bt // GBT
    total_groups = N * n_grp
    per_w = total_groups // num_workers
    idx_per_w = per_w * GBT * BS
    assert per_w * num_workers == total_groups

    @functools.partial(
        pl.kernel,
        out_type=jax.ShapeDtypeStruct((N, D // 8, n_bt, 8, BS), jnp.float32),
        mesh=plsc.VectorSubcoreMesh(core_axis_name="c", subcore_axis_name="s"),
        compiler_params=pltpu.CompilerParams(
            use_tc_tiling_on_sc=False, needs_layout_passes=False
        ),
        scratch_types=[
            pltpu.VMEM((idx_per_w,), jnp.int32),
            pltpu.VMEM((NBUF, CHUNK, D), jnp.float32),
            pltpu.VMEM((D // 8, GBT, 8, BS), jnp.float32),
            pltpu.VMEM((D,), jnp.float32),
        ]
        + [pltpu.SemaphoreType.DMA] * (NBUF + 1),
    )
    def body(idx_hbm, table_hbm, bias_hbm, out_hbm, idx_all, rows, obuf, b_v, *sems):
        gsem = sems[:NBUF]
        osem = sems[NBUF]
        wid = lax.axis_index("s") * info.num_cores + lax.axis_index("c")
        base_g = wid * per_w
        pltpu.sync_copy(bias_hbm, b_v)
        pltpu.sync_copy(idx_hbm.at[pl.ds(base_g * GBT * BS, idx_per_w)], idx_all)

        iota16 = lax.iota(jnp.int32, LANES)

        def gather_start(c, p):
            src = table_hbm.at[idx_all.at[pl.ds(c * CHUNK, CHUNK)]]
            pltpu.async_copy(src, rows.at[p], gsem[p])

        def gather_wait(p):
            src = table_hbm.at[idx_all.at[pl.ds(0, CHUNK)]]
            pltpu.make_async_copy(src, rows.at[p], gsem[p]).wait()

        def store_group(n, bt0):
            for dt in range(D // 8):
                pltpu.async_copy(
                    obuf.at[dt], out_hbm.at[n, dt, pl.ds(bt0, GBT)], osem
                )

        def store_wait_group():
            for dt in range(D // 8):
                pltpu.make_async_copy(
                    obuf.at[dt], out_hbm.at[0, dt, pl.ds(0, GBT)], osem
                ).wait()

        def add_bias(p):
            def one_row(r, _):
                for j in range(D // LANES):
                    sl = pl.ds(j * LANES, LANES)
                    rows[p, r, sl] = rows[p, r, sl] + b_v[sl]
                return 0

            lax.fori_loop(0, CHUNK, one_row, 0)

        def transpose_chunk(p, k):
            @plsc.parallel_loop(0, D, 1, unroll=4)
            def _(col_i):
                dt = col_i // 8
                ds_ = col_i % 8
                col = jnp.full((LANES,), col_i, jnp.int32)
                for btl in range(CHUNK // BS):
                    for g in range(BS // LANES):
                        row_vec = btl * BS + g * LANES + iota16
                        v = plsc.load_gather(rows.at[p], [row_vec, col])
                        obuf[dt, k * 2 + btl, ds_, pl.ds(g * LANES, LANES)] = v

        for p in range(NBUF):
            gather_start(p, p)

        def outer(gi, _):
            g_id = base_g + gi
            n = g_id // n_grp
            bt0 = (g_id % n_grp) * GBT

            @pl.when(gi > 0)
            def _():
                store_wait_group()

            for k in range(CPG):
                c = gi * CPG + k
                p = k % NBUF
                gather_wait(p)

                @pl.when(n % WINDOW == 0)
                def _():
                    add_bias(p)

                transpose_chunk(p, k)

                @pl.when(c + NBUF < per_w * CPG)
                def _():
                    gather_start(c + NBUF, p)

            store_group(n, bt0)
            return 0

        lax.fori_loop(0, per_w, outer, 0)
        store_wait_group()

    out5 = body(idx_t, table, b_embed)
    r = jnp.transpose(out5, (0, 1, 3, 2, 4)).reshape(N, D, B)
    return jnp.transpose(r, (2, 0, 1))


# speedup vs baseline: 1.1179x; 1.1179x over previous
import functools

import jax
import jax.numpy as jnp
from jax import lax
from jax.experimental import pallas as pl
from jax.experimental.pallas import tpu as pltpu
from jax.experimental.pallas import tpu_sc as plsc

D = 64
WINDOW = 8
LANES = 16
CHUNK = 256
NBUF = 4


def kernel(input, table, b_embed):
    B, N = input.shape
    BN = B * N
    idx_flat = input.reshape(BN).astype(jnp.int32)

    info = plsc.get_sparse_core_info()
    num_workers = info.num_cores * info.num_subcores
    per_w = BN // num_workers
    assert per_w * num_workers == BN and per_w % (CHUNK * NBUF) == 0
    n_chunks = per_w // CHUNK
    n_outer = n_chunks // NBUF

    @functools.partial(
        pl.kernel,
        out_type=jax.ShapeDtypeStruct((BN, D), jnp.float32),
        mesh=plsc.VectorSubcoreMesh(core_axis_name="c", subcore_axis_name="s"),
        compiler_params=pltpu.CompilerParams(use_tc_tiling_on_sc=False),
        scratch_types=[
            pltpu.VMEM((per_w,), jnp.int32),
            pltpu.VMEM((NBUF, CHUNK, D), jnp.float32),
            pltpu.VMEM((D,), jnp.float32),
        ]
        + [pltpu.SemaphoreType.DMA] * (2 * NBUF),
    )
    def body(idx_hbm, table_hbm, bias_hbm, out_hbm, idx_all, rows, b_v, *sems):
        gsem = sems[:NBUF]
        ssem = sems[NBUF:]
        wid = lax.axis_index("s") * info.num_cores + lax.axis_index("c")
        base = wid * per_w
        pltpu.sync_copy(bias_hbm, b_v)
        pltpu.sync_copy(idx_hbm.at[pl.ds(base, per_w)], idx_all)

        def gather_start(c, p):
            src = table_hbm.at[idx_all.at[pl.ds(c * CHUNK, CHUNK)]]
            pltpu.async_copy(src, rows.at[p], gsem[p])

        def gather_wait(p):
            src = table_hbm.at[idx_all.at[pl.ds(0, CHUNK)]]
            pltpu.make_async_copy(src, rows.at[p], gsem[p]).wait()

        def store_start(c, p):
            dst = out_hbm.at[pl.ds(base + c * CHUNK, CHUNK)]
            pltpu.async_copy(rows.at[p], dst, ssem[p])

        def store_wait(p):
            dst = out_hbm.at[pl.ds(base, CHUNK)]
            pltpu.make_async_copy(rows.at[p], dst, ssem[p]).wait()

        def add_bias(p):
            def beacon_row(r, _):
                row = r * WINDOW
                for j in range(D // LANES):
                    sl = pl.ds(j * LANES, LANES)
                    rows[p, row, sl] = rows[p, row, sl] + b_v[sl]
                return 0

            lax.fori_loop(0, CHUNK // WINDOW, beacon_row, 0)

        for p in range(NBUF):
            gather_start(p, p)

        def outer(t, _):
            for p in range(NBUF):
                gather_wait(p)
                add_bias(p)
                store_start(t * NBUF + p, p)

            @pl.when(t != n_outer - 1)
            def _prefetch():
                for p in range(NBUF):
                    store_wait(p)
                    gather_start((t + 1) * NBUF + p, p)

            return 0

        lax.fori_loop(0, n_outer, outer, 0)
        for p in range(NBUF):
            store_wait(p)

    out = body(idx_flat, table, b_embed)
    return out.reshape(B, N, D)
